# baseline (speedup 1.0000x reference)
import functools
import jax
import jax.numpy as jnp
from jax import lax
from jax.experimental import pallas as pl
from jax.experimental.pallas import tpu as pltpu

N_NODES = 57000
BLK = 1000
HID = 128
WIDE = 144


def _proj_ab_kernel(x_ref, wp_ref, bp_ref, w1_ref, b1_ref, h_ref, a_ref, b_ref):
    h = jnp.dot(x_ref[...], wp_ref[...], preferred_element_type=jnp.float32) + bp_ref[...]
    h_ref[...] = h
    a = jnp.dot(h, w1_ref[:HID, :], preferred_element_type=jnp.float32) + b1_ref[...]
    b = jnp.dot(h, w1_ref[HID:, :], preferred_element_type=jnp.float32)
    half = jnp.full((h.shape[0], WIDE - HID), 0.5, jnp.float32)
    a_ref[...] = jnp.concatenate([a, half], axis=1)
    b_ref[...] = jnp.concatenate([b, half], axis=1)


def _proj_ab(x, Wp, bp, W1, b1):
    grid = N_NODES // BLK
    return pl.pallas_call(
        _proj_ab_kernel,
        grid=(grid,),
        in_specs=[
            pl.BlockSpec((BLK, HID), lambda i: (i, 0)),
            pl.BlockSpec((HID, HID), lambda i: (0, 0)),
            pl.BlockSpec((HID,), lambda i: (0,)),
            pl.BlockSpec((2 * HID, HID), lambda i: (0, 0)),
            pl.BlockSpec((HID,), lambda i: (0,)),
        ],
        out_specs=[
            pl.BlockSpec((BLK, HID), lambda i: (i, 0)),
            pl.BlockSpec((BLK, WIDE), lambda i: (i, 0)),
            pl.BlockSpec((BLK, WIDE), lambda i: (i, 0)),
        ],
        out_shape=[
            jax.ShapeDtypeStruct((N_NODES, HID), jnp.float32),
            jax.ShapeDtypeStruct((N_NODES, WIDE), jnp.float32),
            jax.ShapeDtypeStruct((N_NODES, WIDE), jnp.float32),
        ],
    )(x, Wp, bp, W1, b1)


def _upd1_kernel(s_ref, h_ref, w2_ref, b2_ref, u1_ref, ub1_ref, u2_ref, ub2_ref,
                 h2_ref, sums_ref):
    s = s_ref[...]
    cnt = jnp.sum(s[:, HID:], axis=1, keepdims=True) * (1.0 / (WIDE - HID))
    aggr = (jnp.dot(s[:, :HID], w2_ref[...], preferred_element_type=jnp.float32)
            + cnt * b2_ref[...]) / jnp.maximum(cnt, 1.0)
    h = h_ref[...]
    p = jnp.dot(h, u1_ref[:HID, :], preferred_element_type=jnp.float32)
    p += jnp.dot(aggr, u1_ref[HID:, :], preferred_element_type=jnp.float32)
    p = jax.nn.relu(p + ub1_ref[...])
    h2 = jnp.dot(p, u2_ref[...], preferred_element_type=jnp.float32) + ub2_ref[...]
    h2_ref[...] = h2
    part = jnp.stack([jnp.sum(h2, axis=0), jnp.sum(h2 * h2, axis=0)])

    @pl.when(pl.program_id(0) == 0)
    def _():
        sums_ref[...] = jnp.zeros_like(sums_ref)

    sums_ref[...] += part


def _upd1(S, h, W2, b2, U1, ub1, U2, ub2):
    grid = N_NODES // BLK
    return pl.pallas_call(
        _upd1_kernel,
        grid=(grid,),
        in_specs=[
            pl.BlockSpec((BLK, WIDE), lambda i: (i, 0)),
            pl.BlockSpec((BLK, HID), lambda i: (i, 0)),
            pl.BlockSpec((HID, HID), lambda i: (0, 0)),
            pl.BlockSpec((HID,), lambda i: (0,)),
            pl.BlockSpec((2 * HID, HID), lambda i: (0, 0)),
            pl.BlockSpec((HID,), lambda i: (0,)),
            pl.BlockSpec((HID, HID), lambda i: (0, 0)),
            pl.BlockSpec((HID,), lambda i: (0,)),
        ],
        out_specs=[
            pl.BlockSpec((BLK, HID), lambda i: (i, 0)),
            pl.BlockSpec((2, HID), lambda i: (0, 0)),
        ],
        out_shape=[
            jax.ShapeDtypeStruct((N_NODES, HID), jnp.float32),
            jax.ShapeDtypeStruct((2, HID), jnp.float32),
        ],
    )(S, h, W2, b2, U1, ub1, U2, ub2)


def _upd2_kernel(h2_ref, sums_ref, hprev_ref, g_ref, b_ref, w1_ref, b1_ref,
                 h_ref, a_ref, bout_ref):
    inv_n = 1.0 / N_NODES
    mean = sums_ref[0:1, :] * inv_n
    var = sums_ref[1:2, :] * inv_n - mean * mean
    rstd = lax.rsqrt(var + 1e-5)
    hb = jax.nn.relu((h2_ref[...] - mean) * rstd * g_ref[...] + b_ref[...])
    h = hb + hprev_ref[...]
    h_ref[...] = h
    a = jnp.dot(h, w1_ref[:HID, :], preferred_element_type=jnp.float32) + b1_ref[...]
    b = jnp.dot(h, w1_ref[HID:, :], preferred_element_type=jnp.float32)
    half = jnp.full((h.shape[0], WIDE - HID), 0.5, jnp.float32)
    a_ref[...] = jnp.concatenate([a, half], axis=1)
    bout_ref[...] = jnp.concatenate([b, half], axis=1)


def _upd2(h2, sums, hprev, g, b, W1n, b1n):
    grid = N_NODES // BLK
    return pl.pallas_call(
        _upd2_kernel,
        grid=(grid,),
        in_specs=[
            pl.BlockSpec((BLK, HID), lambda i: (i, 0)),
            pl.BlockSpec((2, HID), lambda i: (0, 0)),
            pl.BlockSpec((BLK, HID), lambda i: (i, 0)),
            pl.BlockSpec((HID,), lambda i: (0,)),
            pl.BlockSpec((HID,), lambda i: (0,)),
            pl.BlockSpec((2 * HID, HID), lambda i: (0, 0)),
            pl.BlockSpec((HID,), lambda i: (0,)),
        ],
        out_specs=[
            pl.BlockSpec((BLK, HID), lambda i: (i, 0)),
            pl.BlockSpec((BLK, WIDE), lambda i: (i, 0)),
            pl.BlockSpec((BLK, WIDE), lambda i: (i, 0)),
        ],
        out_shape=[
            jax.ShapeDtypeStruct((N_NODES, HID), jnp.float32),
            jax.ShapeDtypeStruct((N_NODES, WIDE), jnp.float32),
            jax.ShapeDtypeStruct((N_NODES, WIDE), jnp.float32),
        ],
    )(h2, sums, hprev, g, b, W1n, b1n)


def _out_kernel(g_ref, w2_ref, b2_ref, o_ref):
    o_ref[...] = (jnp.dot(g_ref[...], w2_ref[...], preferred_element_type=jnp.float32)
                  + b2_ref[...])


def _out_mlp(G, W2pad, b2pad, n_rows, blk):
    grid = n_rows // blk
    return pl.pallas_call(
        _out_kernel,
        grid=(grid,),
        in_specs=[
            pl.BlockSpec((blk, WIDE), lambda i: (i, 0)),
            pl.BlockSpec((WIDE, HID), lambda i: (0, 0)),
            pl.BlockSpec((HID,), lambda i: (0,)),
        ],
        out_specs=pl.BlockSpec((blk, HID), lambda i: (i, 0)),
        out_shape=jax.ShapeDtypeStruct((n_rows, HID), jnp.float32),
    )(G, W2pad, b2pad)


def _edge_stage_xla(A, B, src, dst):
    t = jax.nn.relu(A[dst] + B[src])
    return jax.ops.segment_sum(t, dst, num_segments=N_NODES)


def _final_gather_xla(Pu, Pv, u_idx, v_idx):
    return jax.nn.relu(Pu[u_idx] + Pv[v_idx])


def kernel(x, edge_index, num_graphs, branch_u, branch_v, Wp, bp, msg_W1, msg_b1,
           msg_W2, msg_b2, upd_W1, upd_b1, upd_W2, upd_b2, bn_g, bn_b,
           mlp_W1, mlp_b1, mlp_W2, mlp_b2):
    src = edge_index[0]
    dst = edge_index[1]

    h, A, B = _proj_ab(x, Wp, bp, msg_W1[0], msg_b1[0])
    for l in range(5):
        S = _edge_stage_xla(A, B, src, dst)
        h2, sums = _upd1(S, h, msg_W2[l], msg_b2[l], upd_W1[l], upd_b1[l],
                         upd_W2[l], upd_b2[l])
        if l < 4:
            W1n, b1n = msg_W1[l + 1], msg_b1[l + 1]
        else:
            W1n, b1n = mlp_W1, mlp_b1
        h, A, B = _upd2(h2, sums, h, bn_g[l], bn_b[l], W1n, b1n)

    ng = jnp.asarray(num_graphs, jnp.int32) - jnp.int32(1000)
    offsets = (jnp.arange(1000, dtype=jnp.int32) + ng) * 57
    u_idx = (branch_u[None, :] + offsets[:, None]).reshape(-1)
    v_idx = (branch_v[None, :] + offsets[:, None]).reshape(-1)
    n_out = 80000
    pad = 81920 - n_out
    u_pad = jnp.concatenate([u_idx, jnp.zeros((pad,), jnp.int32)])
    v_pad = jnp.concatenate([v_idx, jnp.zeros((pad,), jnp.int32)])

    G = _final_gather_xla(A, B, u_pad, v_pad)
    W2pad = jnp.zeros((WIDE, HID), jnp.float32).at[:HID, 0].set(mlp_W2[:, 0])
    b2pad = jnp.zeros((HID,), jnp.float32).at[0].set(mlp_b2[0])
    out = _out_mlp(G, W2pad, b2pad, 81920, 8192)
    return out[:n_out, :1]
